# Initial kernel scaffold; baseline (speedup 1.0000x reference)
import jax
import jax.numpy as jnp
from jax.experimental import pallas as pl


def kernel(x, edge_index, edge_attr, params):
    raise NotImplementedError("write your pallas kernel here")


# speedup vs baseline: 3.0903x; 3.0903x over previous
import functools

import jax
import jax.numpy as jnp
from jax import lax
from jax.experimental import pallas as pl
from jax.experimental.pallas import tpu as pltpu
from jax.experimental.pallas import tpu_sc as plsc

N = 10000
E = 160000
LATENT = 128
NUM_MP = 10

CHUNK = 128
NCHUNKS = E // CHUNK
NC, NS = 2, 16
NW = NC * NS
MAXC = (NCHUNKS + NW - 1) // NW
STRIPE = 640
STRIPE_LAST = N - 15 * STRIPE

_mesh = plsc.VectorSubcoreMesh(core_axis_name="c", subcore_axis_name="s")


@functools.partial(
    pl.kernel,
    out_type=(jax.ShapeDtypeStruct((E, LATENT), jnp.float32),
              jax.ShapeDtypeStruct((E, LATENT), jnp.float32)),
    mesh=_mesh,
    scratch_types=[
        pltpu.VMEM((CHUNK,), jnp.int32),
        pltpu.VMEM((CHUNK, LATENT), jnp.float32),
        pltpu.SemaphoreType.DMA,
    ],
)
def _sc_gather(p_hbm, q_hbm, src_hbm, dst_hbm, g1_hbm, g2_hbm, idx_v, rows_v, sem):
    wid = lax.axis_index("s") * NC + lax.axis_index("c")

    def body(i, carry):
        chunk = wid + i * NW

        @pl.when(chunk < NCHUNKS)
        def _():
            base = chunk * CHUNK
            pltpu.sync_copy(src_hbm.at[pl.ds(base, CHUNK)], idx_v)
            pltpu.async_copy(p_hbm.at[idx_v], rows_v, sem).wait()
            pltpu.sync_copy(rows_v, g1_hbm.at[pl.ds(base, CHUNK)])
            pltpu.sync_copy(dst_hbm.at[pl.ds(base, CHUNK)], idx_v)
            pltpu.async_copy(q_hbm.at[idx_v], rows_v, sem).wait()
            pltpu.sync_copy(rows_v, g2_hbm.at[pl.ds(base, CHUNK)])

        return carry

    lax.fori_loop(0, MAXC, body, 0)


@functools.partial(
    pl.kernel,
    out_type=jax.ShapeDtypeStruct((NC, N, LATENT), jnp.float32),
    mesh=_mesh,
    scratch_types=[
        pltpu.VMEM((CHUNK,), jnp.int32),
        pltpu.VMEM((CHUNK, LATENT), jnp.float32),
        pltpu.VMEM_SHARED((N, LATENT), jnp.float32),
        pltpu.SemaphoreType.DMA,
    ],
)
def _sc_scatter(m_hbm, dst_hbm, zeros_hbm, out_hbm, idx_v, rows_v, acc_sh, sem):
    cid = lax.axis_index("c")
    sid = lax.axis_index("s")
    wid = sid * NC + cid
    r0 = sid * STRIPE

    @pl.when(sid < NS - 1)
    def _():
        pltpu.sync_copy(zeros_hbm.at[pl.ds(r0, STRIPE)],
                        acc_sh.at[pl.ds(r0, STRIPE)])

    @pl.when(sid == NS - 1)
    def _():
        pltpu.sync_copy(zeros_hbm.at[pl.ds(r0, STRIPE_LAST)],
                        acc_sh.at[pl.ds(r0, STRIPE_LAST)])

    plsc.subcore_barrier()

    def body(i, carry):
        chunk = wid + i * NW

        @pl.when(chunk < NCHUNKS)
        def _():
            base = chunk * CHUNK
            pltpu.sync_copy(dst_hbm.at[pl.ds(base, CHUNK)], idx_v)
            pltpu.sync_copy(m_hbm.at[pl.ds(base, CHUNK)], rows_v)
            pltpu.sync_copy(rows_v, acc_sh.at[idx_v], add=True)

        return carry

    lax.fori_loop(0, MAXC, body, 0)
    plsc.subcore_barrier()

    @pl.when(sid < NS - 1)
    def _():
        pltpu.sync_copy(acc_sh.at[pl.ds(r0, STRIPE)],
                        out_hbm.at[cid].at[pl.ds(r0, STRIPE)])

    @pl.when(sid == NS - 1)
    def _():
        pltpu.sync_copy(acc_sh.at[pl.ds(r0, STRIPE_LAST)],
                        out_hbm.at[cid].at[pl.ds(r0, STRIPE_LAST)])


def _ln(m, g, b):
    mu = jnp.mean(m, axis=-1, keepdims=True)
    var = jnp.mean((m - mu) ** 2, axis=-1, keepdims=True)
    return g * (m - mu) * lax.rsqrt(var + 1e-5) + b


def _enc_node_body(x_ref, we, be, ws0, wd0, oh, op, oq):
    h = jnp.dot(x_ref[...], we[...], preferred_element_type=jnp.float32) + be[...]
    oh[...] = h
    op[...] = jnp.dot(h, ws0[...], preferred_element_type=jnp.float32)
    oq[...] = jnp.dot(h, wd0[...], preferred_element_type=jnp.float32)


def _enc_edge_body(a_ref, we, be, oe):
    oe[...] = jnp.dot(a_ref[...], we[...], preferred_element_type=jnp.float32) + be[...]


def _edge_body(e_ref, g1_ref, g2_ref, w1e, b1, w2, b2, lng, lnb, om):
    t = jnp.dot(e_ref[...], w1e[...], preferred_element_type=jnp.float32)
    t = t + g1_ref[...] + g2_ref[...] + b1[...]
    t = jnp.maximum(t, 0.0)
    m = jnp.dot(t, w2[...], preferred_element_type=jnp.float32) + b2[...]
    m = jnp.maximum(m, 0.0)
    om[...] = _ln(m, lng[...], lnb[...])


def _node_body(h_ref, p0_ref, p1_ref, u1h, u1p, bu1, u2, bu2, lng, lnb,
               wsn, wdn, oh, op, oq):
    pooled = p0_ref[...] + p1_ref[...]
    t = (jnp.dot(h_ref[...], u1h[...], preferred_element_type=jnp.float32)
         + jnp.dot(pooled, u1p[...], preferred_element_type=jnp.float32)
         + bu1[...])
    t = jnp.maximum(t, 0.0)
    u = jnp.dot(t, u2[...], preferred_element_type=jnp.float32) + bu2[...]
    u = jnp.maximum(u, 0.0)
    hn = h_ref[...] + _ln(u, lng[...], lnb[...])
    oh[...] = hn
    op[...] = jnp.dot(hn, wsn[...], preferred_element_type=jnp.float32)
    oq[...] = jnp.dot(hn, wdn[...], preferred_element_type=jnp.float32)


def _node_final_body(h_ref, p0_ref, p1_ref, u1h, u1p, bu1, u2, bu2, lng, lnb,
                     wdec, bdec, oy):
    pooled = p0_ref[...] + p1_ref[...]
    t = (jnp.dot(h_ref[...], u1h[...], preferred_element_type=jnp.float32)
         + jnp.dot(pooled, u1p[...], preferred_element_type=jnp.float32)
         + bu1[...])
    t = jnp.maximum(t, 0.0)
    u = jnp.dot(t, u2[...], preferred_element_type=jnp.float32) + bu2[...]
    u = jnp.maximum(u, 0.0)
    hn = h_ref[...] + _ln(u, lng[...], lnb[...])
    oy[...] = jnp.dot(hn, wdec[...], preferred_element_type=jnp.float32) + bdec[...]


def _row_spec(bm, width):
    return pl.BlockSpec((bm, width), lambda i: (i, 0))


def _full_spec(shape):
    return pl.BlockSpec(shape, lambda i: tuple(0 for _ in shape))


BN = 2000
BE = 4000


def _enc_node_call(x, we, be, ws0, wd0):
    return pl.pallas_call(
        _enc_node_body,
        grid=(N // BN,),
        in_specs=[_row_spec(BN, 256), _full_spec((256, LATENT)),
                  _full_spec((1, LATENT)), _full_spec((LATENT, LATENT)),
                  _full_spec((LATENT, LATENT))],
        out_specs=[_row_spec(BN, LATENT)] * 3,
        out_shape=[jax.ShapeDtypeStruct((N, LATENT), jnp.float32)] * 3,
    )(x, we, be, ws0, wd0)


def _enc_edge_call(attr, we, be):
    return pl.pallas_call(
        _enc_edge_body,
        grid=(E // BE,),
        in_specs=[_row_spec(BE, 16), _full_spec((16, LATENT)),
                  _full_spec((1, LATENT))],
        out_specs=_row_spec(BE, LATENT),
        out_shape=jax.ShapeDtypeStruct((E, LATENT), jnp.float32),
    )(attr, we, be)


def _edge_call(e, g1, g2, w1e, b1, w2, b2, lng, lnb):
    w = _full_spec((LATENT, LATENT))
    v = _full_spec((1, LATENT))
    return pl.pallas_call(
        _edge_body,
        grid=(E // BE,),
        in_specs=[_row_spec(BE, LATENT)] * 3 + [w, v, w, v, v, v],
        out_specs=_row_spec(BE, LATENT),
        out_shape=jax.ShapeDtypeStruct((E, LATENT), jnp.float32),
    )(e, g1, g2, w1e, b1, w2, b2, lng, lnb)


def _node_call(h, p0, p1, u1h, u1p, bu1, u2, bu2, lng, lnb, wsn, wdn):
    w = _full_spec((LATENT, LATENT))
    v = _full_spec((1, LATENT))
    return pl.pallas_call(
        _node_body,
        grid=(N // BN,),
        in_specs=[_row_spec(BN, LATENT)] * 3 + [w, w, v, w, v, v, v, w, w],
        out_specs=[_row_spec(BN, LATENT)] * 3,
        out_shape=[jax.ShapeDtypeStruct((N, LATENT), jnp.float32)] * 3,
    )(h, p0, p1, u1h, u1p, bu1, u2, bu2, lng, lnb, wsn, wdn)


def _node_final_call(h, p0, p1, u1h, u1p, bu1, u2, bu2, lng, lnb, wdec, bdec):
    w = _full_spec((LATENT, LATENT))
    v = _full_spec((1, LATENT))
    return pl.pallas_call(
        _node_final_body,
        grid=(N // BN,),
        in_specs=[_row_spec(BN, LATENT)] * 3 + [w, w, v, w, v, v, v, w, v],
        out_specs=_row_spec(BN, LATENT),
        out_shape=jax.ShapeDtypeStruct((N, LATENT), jnp.float32),
    )(h, p0, p1, u1h, u1p, bu1, u2, bu2, lng, lnb, wdec, bdec)


def kernel(x, edge_index, edge_attr, params):
    src = edge_index[0]
    dst = edge_index[1]
    layers = params["layers"]

    def row(v):
        return v.reshape(1, -1)

    w1 = [lp["msg1"]["w"] for lp in layers]
    w1e = [w[0:LATENT] for w in w1]
    w1s = [w[LATENT:2 * LATENT] for w in w1]
    w1d = [w[2 * LATENT:3 * LATENT] for w in w1]
    u1 = [lp["upd1"]["w"] for lp in layers]
    u1h = [w[0:LATENT] for w in u1]
    u1p = [w[LATENT:2 * LATENT] for w in u1]

    wdec = jnp.zeros((LATENT, LATENT), jnp.float32).at[:, :3].set(params["dec"]["w"])
    bdec = jnp.zeros((1, LATENT), jnp.float32).at[:, :3].set(params["dec"]["b"])
    zeros_n = jnp.zeros((N, LATENT), jnp.float32)

    h, p, q = _enc_node_call(x, params["enc_node"]["w"], row(params["enc_node"]["b"]),
                             w1s[0], w1d[0])
    e = _enc_edge_call(edge_attr, params["enc_edge"]["w"], row(params["enc_edge"]["b"]))

    for l in range(NUM_MP):
        lp = layers[l]
        g1, g2 = _sc_gather(p, q, src, dst)
        m = _edge_call(e, g1, g2, w1e[l], row(lp["msg1"]["b"]),
                       lp["msg2"]["w"], row(lp["msg2"]["b"]),
                       row(lp["msg_ln_g"]), row(lp["msg_ln_b"]))
        pooled2 = _sc_scatter(m, dst, zeros_n)
        args = (h, pooled2[0], pooled2[1], u1h[l], u1p[l], row(lp["upd1"]["b"]),
                lp["upd2"]["w"], row(lp["upd2"]["b"]),
                row(lp["upd_ln_g"]), row(lp["upd_ln_b"]))
        if l < NUM_MP - 1:
            h, p, q = _node_call(*args, w1s[l + 1], w1d[l + 1])
        else:
            y = _node_final_call(*args, wdec, bdec)

    return y[:, :3]


# speedup vs baseline: 4.3553x; 1.4094x over previous
import functools

import jax
import jax.numpy as jnp
from jax import lax
from jax.experimental import pallas as pl
from jax.experimental.pallas import tpu as pltpu
from jax.experimental.pallas import tpu_sc as plsc

N = 10000
E = 160000
LATENT = 128
NUM_MP = 10

CHUNK = 128
NCHUNKS = E // CHUNK
NC, NS = 2, 16
NW = NC * NS
WCH = NCHUNKS // NW
NREM = NCHUNKS - WCH * NW
STRIPE = 640
STRIPE_LAST = N - 15 * STRIPE

_mesh = plsc.VectorSubcoreMesh(core_axis_name="c", subcore_axis_name="s")


@functools.partial(
    pl.kernel,
    out_type=(jax.ShapeDtypeStruct((E, LATENT), jnp.float32),
              jax.ShapeDtypeStruct((E, LATENT), jnp.float32)),
    mesh=_mesh,
    scratch_types=[
        pltpu.VMEM((2, CHUNK), jnp.int32),
        pltpu.VMEM((2, CHUNK), jnp.int32),
        pltpu.VMEM((CHUNK, LATENT), jnp.float32),
        pltpu.VMEM((CHUNK, LATENT), jnp.float32),
        pltpu.VMEM((CHUNK, LATENT), jnp.float32),
        pltpu.VMEM((CHUNK, LATENT), jnp.float32),
        pltpu.SemaphoreType.DMA,
        pltpu.SemaphoreType.DMA,
        pltpu.SemaphoreType.DMA,
        pltpu.SemaphoreType.DMA,
    ],
)
def _sc_gather(pq_hbm, sd_hbm, g1_hbm, g2_hbm,
               idx0, idx1, rp0, rp1, rq0, rq1, sg0, sg1, so0, so1):
    wid = lax.axis_index("s") * NC + lax.axis_index("c")
    c0 = wid * WCH
    idx = (idx0, idx1)
    rp = (rp0, rp1)
    rq = (rq0, rq1)
    sg = (sg0, sg1)
    so = (so0, so1)

    def idx_load(j, b):
        pltpu.sync_copy(sd_hbm.at[c0 + j], idx[b])

    def gathers_start(j, b):
        pltpu.async_copy(pq_hbm.at[idx[b].at[0]], rp[b], sg[b])
        pltpu.async_copy(pq_hbm.at[idx[b].at[1]], rq[b], sg[b])

    def gathers_wait(b):
        pltpu.make_async_copy(pq_hbm.at[idx[b].at[0]], rp[b], sg[b]).wait()
        pltpu.make_async_copy(pq_hbm.at[idx[b].at[1]], rq[b], sg[b]).wait()

    def stores_start(j, b):
        base = (c0 + j) * CHUNK
        pltpu.async_copy(rp[b], g1_hbm.at[pl.ds(base, CHUNK)], so[b])
        pltpu.async_copy(rq[b], g2_hbm.at[pl.ds(base, CHUNK)], so[b])

    def stores_wait(j, b):
        base = (c0 + j) * CHUNK
        pltpu.make_async_copy(rp[b], g1_hbm.at[pl.ds(base, CHUNK)], so[b]).wait()
        pltpu.make_async_copy(rq[b], g2_hbm.at[pl.ds(base, CHUNK)], so[b]).wait()

    idx_load(0, 0)
    gathers_start(0, 0)

    def body(t, carry):
        j0 = 2 * t
        idx_load(j0 + 1, 1)
        gathers_wait(0)

        @pl.when(t > 0)
        def _():
            stores_wait(j0 - 1, 1)

        gathers_start(j0 + 1, 1)
        stores_start(j0, 0)
        idx_load(j0 + 2, 0)
        gathers_wait(1)
        stores_wait(j0, 0)
        gathers_start(j0 + 2, 0)
        stores_start(j0 + 1, 1)
        return carry

    lax.fori_loop(0, (WCH - 1) // 2, body, 0)
    stores_wait(WCH - 2, 1)
    gathers_wait(0)
    stores_start(WCH - 1, 0)
    stores_wait(WCH - 1, 0)

    @pl.when(wid < NREM)
    def _():
        cr = WCH * NW + wid
        pltpu.sync_copy(sd_hbm.at[cr], idx0)
        pltpu.async_copy(pq_hbm.at[idx0.at[0]], rp0, sg0)
        pltpu.async_copy(pq_hbm.at[idx0.at[1]], rq0, sg0)
        pltpu.make_async_copy(pq_hbm.at[idx0.at[0]], rp0, sg0).wait()
        pltpu.make_async_copy(pq_hbm.at[idx0.at[1]], rq0, sg0).wait()
        pltpu.sync_copy(rp0, g1_hbm.at[pl.ds(cr * CHUNK, CHUNK)])
        pltpu.sync_copy(rq0, g2_hbm.at[pl.ds(cr * CHUNK, CHUNK)])


@functools.partial(
    pl.kernel,
    out_type=jax.ShapeDtypeStruct((NC, N, LATENT), jnp.float32),
    mesh=_mesh,
    scratch_types=[
        pltpu.VMEM((CHUNK,), jnp.int32),
        pltpu.VMEM((CHUNK,), jnp.int32),
        pltpu.VMEM((CHUNK, LATENT), jnp.float32),
        pltpu.VMEM((CHUNK, LATENT), jnp.float32),
        pltpu.VMEM_SHARED((N, LATENT), jnp.float32),
        pltpu.SemaphoreType.DMA,
        pltpu.SemaphoreType.DMA,
        pltpu.SemaphoreType.DMA,
        pltpu.SemaphoreType.DMA,
    ],
)
def _sc_scatter(m_hbm, dst_hbm, zeros_hbm, out_hbm,
                idx0, idx1, rm0, rm1, acc_sh, sl0, sl1, ss0, ss1):
    cid = lax.axis_index("c")
    sid = lax.axis_index("s")
    wid = sid * NC + cid
    c0 = wid * WCH
    r0 = sid * STRIPE
    idx = (idx0, idx1)
    rm = (rm0, rm1)
    sl = (sl0, sl1)
    ss = (ss0, ss1)

    @pl.when(sid < NS - 1)
    def _():
        pltpu.sync_copy(zeros_hbm.at[pl.ds(r0, STRIPE)],
                        acc_sh.at[pl.ds(r0, STRIPE)])

    @pl.when(sid == NS - 1)
    def _():
        pltpu.sync_copy(zeros_hbm.at[pl.ds(r0, STRIPE_LAST)],
                        acc_sh.at[pl.ds(r0, STRIPE_LAST)])

    plsc.subcore_barrier()

    def load_start(j, b):
        base = (c0 + j) * CHUNK
        pltpu.async_copy(m_hbm.at[pl.ds(base, CHUNK)], rm[b], sl[b])
        pltpu.sync_copy(dst_hbm.at[pl.ds(base, CHUNK)], idx[b])

    def load_wait(j, b):
        base = (c0 + j) * CHUNK
        pltpu.make_async_copy(m_hbm.at[pl.ds(base, CHUNK)], rm[b], sl[b]).wait()

    def scat_start(b):
        pltpu.async_copy(rm[b], acc_sh.at[idx[b]], ss[b], add=True)

    def scat_wait(b):
        pltpu.make_async_copy(rm[b], acc_sh.at[idx[b]], ss[b]).wait()

    load_start(0, 0)

    def body(t, carry):
        j0 = 2 * t

        @pl.when(t > 0)
        def _():
            scat_wait(1)

        load_start(j0 + 1, 1)
        load_wait(j0, 0)
        scat_start(0)
        load_wait(j0 + 1, 1)
        scat_start(1)
        scat_wait(0)
        load_start(j0 + 2, 0)
        return carry

    lax.fori_loop(0, (WCH - 1) // 2, body, 0)
    scat_wait(1)
    load_wait(WCH - 1, 0)
    scat_start(0)
    scat_wait(0)

    @pl.when(wid < NREM)
    def _():
        cr = WCH * NW + wid
        base = cr * CHUNK
        pltpu.sync_copy(m_hbm.at[pl.ds(base, CHUNK)], rm0)
        pltpu.sync_copy(dst_hbm.at[pl.ds(base, CHUNK)], idx0)
        pltpu.sync_copy(rm0, acc_sh.at[idx0], add=True)

    plsc.subcore_barrier()

    @pl.when(sid < NS - 1)
    def _():
        pltpu.sync_copy(acc_sh.at[pl.ds(r0, STRIPE)],
                        out_hbm.at[cid].at[pl.ds(r0, STRIPE)])

    @pl.when(sid == NS - 1)
    def _():
        pltpu.sync_copy(acc_sh.at[pl.ds(r0, STRIPE_LAST)],
                        out_hbm.at[cid].at[pl.ds(r0, STRIPE_LAST)])


def _ln(m, g, b):
    mu = jnp.mean(m, axis=-1, keepdims=True)
    var = jnp.mean((m - mu) ** 2, axis=-1, keepdims=True)
    return g * (m - mu) * lax.rsqrt(var + 1e-5) + b


def _enc_node_body(x_ref, we, be, ws0, wd0, oh, opq):
    h = jnp.dot(x_ref[...], we[...], preferred_element_type=jnp.float32) + be[...]
    oh[...] = h
    opq[0, :, :] = jnp.dot(h, ws0[...], preferred_element_type=jnp.float32)
    opq[1, :, :] = jnp.dot(h, wd0[...], preferred_element_type=jnp.float32)


def _enc_edge_body(a_ref, we, be, oe):
    oe[...] = jnp.dot(a_ref[...], we[...], preferred_element_type=jnp.float32) + be[...]


def _edge_body(e_ref, g1_ref, g2_ref, w1e, b1, w2, b2, lng, lnb, om):
    t = jnp.dot(e_ref[...], w1e[...], preferred_element_type=jnp.float32)
    t = t + g1_ref[...] + g2_ref[...] + b1[...]
    t = jnp.maximum(t, 0.0)
    m = jnp.dot(t, w2[...], preferred_element_type=jnp.float32) + b2[...]
    m = jnp.maximum(m, 0.0)
    om[...] = _ln(m, lng[...], lnb[...])


def _node_body(h_ref, p0_ref, p1_ref, u1h, u1p, bu1, u2, bu2, lng, lnb,
               wsn, wdn, oh, opq):
    pooled = p0_ref[...] + p1_ref[...]
    t = (jnp.dot(h_ref[...], u1h[...], preferred_element_type=jnp.float32)
         + jnp.dot(pooled, u1p[...], preferred_element_type=jnp.float32)
         + bu1[...])
    t = jnp.maximum(t, 0.0)
    u = jnp.dot(t, u2[...], preferred_element_type=jnp.float32) + bu2[...]
    u = jnp.maximum(u, 0.0)
    hn = h_ref[...] + _ln(u, lng[...], lnb[...])
    oh[...] = hn
    opq[0, :, :] = jnp.dot(hn, wsn[...], preferred_element_type=jnp.float32)
    opq[1, :, :] = jnp.dot(hn, wdn[...], preferred_element_type=jnp.float32)


def _node_final_body(h_ref, p0_ref, p1_ref, u1h, u1p, bu1, u2, bu2, lng, lnb,
                     wdec, bdec, oy):
    pooled = p0_ref[...] + p1_ref[...]
    t = (jnp.dot(h_ref[...], u1h[...], preferred_element_type=jnp.float32)
         + jnp.dot(pooled, u1p[...], preferred_element_type=jnp.float32)
         + bu1[...])
    t = jnp.maximum(t, 0.0)
    u = jnp.dot(t, u2[...], preferred_element_type=jnp.float32) + bu2[...]
    u = jnp.maximum(u, 0.0)
    hn = h_ref[...] + _ln(u, lng[...], lnb[...])
    oy[...] = jnp.dot(hn, wdec[...], preferred_element_type=jnp.float32) + bdec[...]


def _row_spec(bm, width):
    return pl.BlockSpec((bm, width), lambda i: (i, 0))


def _full_spec(shape):
    return pl.BlockSpec(shape, lambda i: tuple(0 for _ in shape))


BN = 2000
BE = 4000


_PQ_SPEC = pl.BlockSpec((2, BN, LATENT), lambda i: (0, i, 0))


def _enc_node_call(x, we, be, ws0, wd0):
    return pl.pallas_call(
        _enc_node_body,
        grid=(N // BN,),
        in_specs=[_row_spec(BN, 256), _full_spec((256, LATENT)),
                  _full_spec((1, LATENT)), _full_spec((LATENT, LATENT)),
                  _full_spec((LATENT, LATENT))],
        out_specs=[_row_spec(BN, LATENT), _PQ_SPEC],
        out_shape=[jax.ShapeDtypeStruct((N, LATENT), jnp.float32),
                   jax.ShapeDtypeStruct((2, N, LATENT), jnp.float32)],
    )(x, we, be, ws0, wd0)


def _enc_edge_call(attr, we, be):
    return pl.pallas_call(
        _enc_edge_body,
        grid=(E // BE,),
        in_specs=[_row_spec(BE, 16), _full_spec((16, LATENT)),
                  _full_spec((1, LATENT))],
        out_specs=_row_spec(BE, LATENT),
        out_shape=jax.ShapeDtypeStruct((E, LATENT), jnp.float32),
    )(attr, we, be)


def _edge_call(e, g1, g2, w1e, b1, w2, b2, lng, lnb):
    w = _full_spec((LATENT, LATENT))
    v = _full_spec((1, LATENT))
    return pl.pallas_call(
        _edge_body,
        grid=(E // BE,),
        in_specs=[_row_spec(BE, LATENT)] * 3 + [w, v, w, v, v, v],
        out_specs=_row_spec(BE, LATENT),
        out_shape=jax.ShapeDtypeStruct((E, LATENT), jnp.float32),
    )(e, g1, g2, w1e, b1, w2, b2, lng, lnb)


def _node_call(h, p0, p1, u1h, u1p, bu1, u2, bu2, lng, lnb, wsn, wdn):
    w = _full_spec((LATENT, LATENT))
    v = _full_spec((1, LATENT))
    return pl.pallas_call(
        _node_body,
        grid=(N // BN,),
        in_specs=[_row_spec(BN, LATENT)] * 3 + [w, w, v, w, v, v, v, w, w],
        out_specs=[_row_spec(BN, LATENT), _PQ_SPEC],
        out_shape=[jax.ShapeDtypeStruct((N, LATENT), jnp.float32),
                   jax.ShapeDtypeStruct((2, N, LATENT), jnp.float32)],
    )(h, p0, p1, u1h, u1p, bu1, u2, bu2, lng, lnb, wsn, wdn)


def _node_final_call(h, p0, p1, u1h, u1p, bu1, u2, bu2, lng, lnb, wdec, bdec):
    w = _full_spec((LATENT, LATENT))
    v = _full_spec((1, LATENT))
    return pl.pallas_call(
        _node_final_body,
        grid=(N // BN,),
        in_specs=[_row_spec(BN, LATENT)] * 3 + [w, w, v, w, v, v, v, w, v],
        out_specs=_row_spec(BN, LATENT),
        out_shape=jax.ShapeDtypeStruct((N, LATENT), jnp.float32),
    )(h, p0, p1, u1h, u1p, bu1, u2, bu2, lng, lnb, wdec, bdec)


def kernel(x, edge_index, edge_attr, params):
    src = edge_index[0]
    dst = edge_index[1]
    sd = jnp.stack([src, dst + N], axis=0).reshape(2, NCHUNKS, CHUNK)
    sd = sd.transpose(1, 0, 2)
    layers = params["layers"]

    def row(v):
        return v.reshape(1, -1)

    w1 = [lp["msg1"]["w"] for lp in layers]
    w1e = [w[0:LATENT] for w in w1]
    w1s = [w[LATENT:2 * LATENT] for w in w1]
    w1d = [w[2 * LATENT:3 * LATENT] for w in w1]
    u1 = [lp["upd1"]["w"] for lp in layers]
    u1h = [w[0:LATENT] for w in u1]
    u1p = [w[LATENT:2 * LATENT] for w in u1]

    wdec = jnp.zeros((LATENT, LATENT), jnp.float32).at[:, :3].set(params["dec"]["w"])
    bdec = jnp.zeros((1, LATENT), jnp.float32).at[:, :3].set(params["dec"]["b"])
    zeros_n = jnp.zeros((N, LATENT), jnp.float32)

    h, pq = _enc_node_call(x, params["enc_node"]["w"], row(params["enc_node"]["b"]),
                           w1s[0], w1d[0])
    e = _enc_edge_call(edge_attr, params["enc_edge"]["w"], row(params["enc_edge"]["b"]))

    for l in range(NUM_MP):
        lp = layers[l]
        g1, g2 = _sc_gather(pq.reshape(2 * N, LATENT), sd)
        m = _edge_call(e, g1, g2, w1e[l], row(lp["msg1"]["b"]),
                       lp["msg2"]["w"], row(lp["msg2"]["b"]),
                       row(lp["msg_ln_g"]), row(lp["msg_ln_b"]))
        pooled2 = _sc_scatter(m, dst, zeros_n)
        args = (h, pooled2[0], pooled2[1], u1h[l], u1p[l], row(lp["upd1"]["b"]),
                lp["upd2"]["w"], row(lp["upd2"]["b"]),
                row(lp["upd_ln_g"]), row(lp["upd_ln_b"]))
        if l < NUM_MP - 1:
            h, pq = _node_call(*args, w1s[l + 1], w1d[l + 1])
        else:
            y = _node_final_call(*args, wdec, bdec)

    return y[:, :3]
